# baseline (speedup 1.0000x reference)
import jax
import jax.numpy as jnp
from jax.experimental import pallas as pl
from jax.experimental.pallas import tpu as pltpu

_B = 1024
_NF = 100000
_D = 64
_H1 = 64
_H2 = 32
_KT = 1000
_NK = _NF // _KT
_EPS = 1e-5


def _bn(v, g, b):
    mu = jnp.mean(v, axis=0, keepdims=True)
    var = jnp.mean(jnp.square(v - mu), axis=0, keepdims=True)
    return (v - mu) / jnp.sqrt(var + _EPS) * g + b


def _nfm_kernel(x_ref, e_ref, lw_ref, lb_ref, g0_ref, b0_ref,
                w1_ref, b1_ref, g1_ref, bb1_ref,
                w2_ref, b2_ref, g2_ref, bb2_ref, hw_ref,
                out_ref, acc_a, acc_q):
    k = pl.program_id(0)

    @pl.when(k == 0)
    def _init():
        acc_a[...] = jnp.zeros_like(acc_a)
        acc_q[...] = jnp.zeros_like(acc_q)

    x = x_ref[:, 0, 0, :].astype(jnp.bfloat16)
    e = e_ref[0].astype(jnp.bfloat16)
    lw = lw_ref[0].astype(jnp.bfloat16)
    aug = jnp.concatenate([e, lw], axis=1)
    acc_a[...] += jnp.dot(x, aug, preferred_element_type=jnp.float32)
    acc_q[...] += jnp.dot(x * x, e * e, preferred_element_type=jnp.float32)

    @pl.when(k == _NK - 1)
    def _epilogue():
        se = acc_a[:, :_D]
        col = jax.lax.broadcasted_iota(jnp.int32, (1, _D + 1), 1)
        onehot = jnp.where(col == _D, 1.0, 0.0).astype(jnp.float32)
        lin = jnp.sum(acc_a[...] * onehot, axis=1, keepdims=True)
        bi = 0.5 * (se * se - acc_q[...])
        z = _bn(bi, g0_ref[...], b0_ref[...])
        z = jax.lax.dot_general(z, w1_ref[...], (((1,), (1,)), ((), ())),
                                preferred_element_type=jnp.float32) + b1_ref[...]
        z = jax.nn.relu(_bn(z, g1_ref[...], bb1_ref[...]))
        z = jax.lax.dot_general(z, w2_ref[...], (((1,), (1,)), ((), ())),
                                preferred_element_type=jnp.float32) + b2_ref[...]
        z = jax.nn.relu(_bn(z, g2_ref[...], bb2_ref[...]))
        y = jnp.sum(z * hw_ref[...], axis=1, keepdims=True)
        out_ref[...] = y + lin + lb_ref[...]


def kernel(feature_values, feature_embed, lin_w, lin_b, bn0_g, bn0_b,
           W1, b1, bn1_g, bn1_b, W2, b2, bn2_g, bn2_b, h_w):
    fv4 = feature_values.reshape(_B, _NK, 1, _KT)
    e3 = feature_embed.reshape(_NK, _KT, _D)
    lw3 = lin_w.reshape(_NK, _KT, 1)
    out = pl.pallas_call(
        _nfm_kernel,
        grid=(_NK,),
        in_specs=[
            pl.BlockSpec((_B, 1, 1, _KT), lambda k: (0, k, 0, 0)),
            pl.BlockSpec((1, _KT, _D), lambda k: (k, 0, 0)),
            pl.BlockSpec((1, _KT, 1), lambda k: (k, 0, 0)),
            pl.BlockSpec((1, 1), lambda k: (0, 0)),
            pl.BlockSpec((1, _D), lambda k: (0, 0)),
            pl.BlockSpec((1, _D), lambda k: (0, 0)),
            pl.BlockSpec((_H1, _D), lambda k: (0, 0)),
            pl.BlockSpec((1, _H1), lambda k: (0, 0)),
            pl.BlockSpec((1, _H1), lambda k: (0, 0)),
            pl.BlockSpec((1, _H1), lambda k: (0, 0)),
            pl.BlockSpec((_H2, _H1), lambda k: (0, 0)),
            pl.BlockSpec((1, _H2), lambda k: (0, 0)),
            pl.BlockSpec((1, _H2), lambda k: (0, 0)),
            pl.BlockSpec((1, _H2), lambda k: (0, 0)),
            pl.BlockSpec((1, _H2), lambda k: (0, 0)),
        ],
        out_specs=pl.BlockSpec((_B, 1), lambda k: (0, 0)),
        out_shape=jax.ShapeDtypeStruct((_B, 1), jnp.float32),
        scratch_shapes=[
            pltpu.VMEM((_B, _D + 1), jnp.float32),
            pltpu.VMEM((_B, _D), jnp.float32),
        ],
        compiler_params=pltpu.CompilerParams(
            dimension_semantics=("arbitrary",),
        ),
    )(fv4, e3, lw3,
      lin_b.reshape(1, 1), bn0_g.reshape(1, _D), bn0_b.reshape(1, _D),
      W1, b1.reshape(1, _H1), bn1_g.reshape(1, _H1), bn1_b.reshape(1, _H1),
      W2, b2.reshape(1, _H2), bn2_g.reshape(1, _H2), bn2_b.reshape(1, _H2),
      h_w)
    return out.reshape(_B)


# speedup vs baseline: 3.1892x; 3.1892x over previous
import jax
import jax.numpy as jnp
from jax.experimental import pallas as pl
from jax.experimental.pallas import tpu as pltpu

_B = 1024
_NF = 100000
_D = 64
_H1 = 64
_H2 = 32
_KT = 2048
_NB = (_NF + _KT - 1) // _KT
_EPS = 1e-5


def _bn(v, g, b):
    mu = jnp.mean(v, axis=0, keepdims=True)
    var = jnp.mean(jnp.square(v - mu), axis=0, keepdims=True)
    return (v - mu) / jnp.sqrt(var + _EPS) * g + b


def _nfm_kernel(x_ref, e_ref, lw_ref, lb_ref, g0_ref, b0_ref,
                w1_ref, b1_ref, g1_ref, bb1_ref,
                w2_ref, b2_ref, g2_ref, bb2_ref, hw_ref,
                out_ref, acc_a, acc_q):
    k = pl.program_id(0)

    @pl.when(k == 0)
    def _init():
        acc_a[...] = jnp.zeros_like(acc_a)
        acc_q[...] = jnp.zeros_like(acc_q)

    def _accumulate(x, e, lw):
        aug = jnp.concatenate([e, lw], axis=1)
        acc_a[...] += jnp.dot(x, aug, preferred_element_type=jnp.float32)
        acc_q[...] += jnp.dot(x * x, e * e, preferred_element_type=jnp.float32)

    @pl.when(k < _NB - 1)
    def _full_tile():
        _accumulate(x_ref[...].astype(jnp.bfloat16),
                    e_ref[...].astype(jnp.bfloat16),
                    lw_ref[...].astype(jnp.bfloat16))

    @pl.when(k == _NB - 1)
    def _partial_tile():
        nvalid = _NF - (_NB - 1) * _KT
        lane = jax.lax.broadcasted_iota(jnp.int32, (1, _KT), 1)
        sub = jax.lax.broadcasted_iota(jnp.int32, (_KT, 1), 0)
        x = jnp.where(lane < nvalid, x_ref[...], 0.0).astype(jnp.bfloat16)
        e = jnp.where(sub < nvalid, e_ref[...], 0.0).astype(jnp.bfloat16)
        lw = jnp.where(sub < nvalid, lw_ref[...], 0.0).astype(jnp.bfloat16)
        _accumulate(x, e, lw)

        se = acc_a[:, :_D]
        col = jax.lax.broadcasted_iota(jnp.int32, (1, _D + 1), 1)
        onehot = jnp.where(col == _D, 1.0, 0.0).astype(jnp.float32)
        lin = jnp.sum(acc_a[...] * onehot, axis=1, keepdims=True)
        bi = 0.5 * (se * se - acc_q[...])
        z = _bn(bi, g0_ref[...], b0_ref[...])
        z = jax.lax.dot_general(z, w1_ref[...], (((1,), (1,)), ((), ())),
                                preferred_element_type=jnp.float32) + b1_ref[...]
        z = jax.nn.relu(_bn(z, g1_ref[...], bb1_ref[...]))
        z = jax.lax.dot_general(z, w2_ref[...], (((1,), (1,)), ((), ())),
                                preferred_element_type=jnp.float32) + b2_ref[...]
        z = jax.nn.relu(_bn(z, g2_ref[...], bb2_ref[...]))
        y = jnp.sum(z * hw_ref[...], axis=1, keepdims=True)
        out_ref[...] = y + lin + lb_ref[...]


def kernel(feature_values, feature_embed, lin_w, lin_b, bn0_g, bn0_b,
           W1, b1, bn1_g, bn1_b, W2, b2, bn2_g, bn2_b, h_w):
    out = pl.pallas_call(
        _nfm_kernel,
        grid=(_NB,),
        in_specs=[
            pl.BlockSpec((_B, _KT), lambda k: (0, k)),
            pl.BlockSpec((_KT, _D), lambda k: (k, 0)),
            pl.BlockSpec((_KT, 1), lambda k: (k, 0)),
            pl.BlockSpec((1, 1), lambda k: (0, 0)),
            pl.BlockSpec((1, _D), lambda k: (0, 0)),
            pl.BlockSpec((1, _D), lambda k: (0, 0)),
            pl.BlockSpec((_H1, _D), lambda k: (0, 0)),
            pl.BlockSpec((1, _H1), lambda k: (0, 0)),
            pl.BlockSpec((1, _H1), lambda k: (0, 0)),
            pl.BlockSpec((1, _H1), lambda k: (0, 0)),
            pl.BlockSpec((_H2, _H1), lambda k: (0, 0)),
            pl.BlockSpec((1, _H2), lambda k: (0, 0)),
            pl.BlockSpec((1, _H2), lambda k: (0, 0)),
            pl.BlockSpec((1, _H2), lambda k: (0, 0)),
            pl.BlockSpec((1, _H2), lambda k: (0, 0)),
        ],
        out_specs=pl.BlockSpec((_B, 1), lambda k: (0, 0)),
        out_shape=jax.ShapeDtypeStruct((_B, 1), jnp.float32),
        scratch_shapes=[
            pltpu.VMEM((_B, _D + 1), jnp.float32),
            pltpu.VMEM((_B, _D), jnp.float32),
        ],
        compiler_params=pltpu.CompilerParams(
            dimension_semantics=("arbitrary",),
        ),
    )(feature_values, feature_embed, lin_w.reshape(_NF, 1),
      lin_b.reshape(1, 1), bn0_g.reshape(1, _D), bn0_b.reshape(1, _D),
      W1, b1.reshape(1, _H1), bn1_g.reshape(1, _H1), bn1_b.reshape(1, _H1),
      W2, b2.reshape(1, _H2), bn2_g.reshape(1, _H2), bn2_b.reshape(1, _H2),
      h_w)
    return out.reshape(_B)
